# baseline (speedup 1.0000x reference)
import functools

import jax
import jax.numpy as jnp
from jax.experimental import pallas as pl
from jax.experimental.pallas import tpu as pltpu

_B, _T, _P, _F = 16, 50, 200, 128
_N = _B * _T
_G = 8


def _tc_body(tl_sp, tt_sp, x_ref, pt_ref, ft_ref, o_ref):
    g = pl.program_id(0)
    pf = pt_ref[...] + ft_ref[...]
    for r in range(_G):
        idx = tl_sp[g * _G + r] // 3
        te = tt_sp[idx]
        o_ref[r] = x_ref[r] + (pf + te)


@jax.jit
def kernel(x, time_list, time_table, point_table, f_table):
    x3 = x.reshape(_N, _P, _F)
    tl = time_list.reshape(_N).astype(jnp.int32)
    tt = time_table.reshape(-1)
    pt = point_table.reshape(_P, 1)
    ft = f_table.reshape(1, _F)

    grid_spec = pltpu.PrefetchScalarGridSpec(
        num_scalar_prefetch=2,
        grid=(_N // _G,),
        in_specs=[
            pl.BlockSpec((_G, _P, _F), lambda g, tl_sp, tt_sp: (g, 0, 0)),
            pl.BlockSpec((_P, 1), lambda g, tl_sp, tt_sp: (0, 0)),
            pl.BlockSpec((1, _F), lambda g, tl_sp, tt_sp: (0, 0)),
        ],
        out_specs=pl.BlockSpec((_G, _P, _F), lambda g, tl_sp, tt_sp: (g, 0, 0)),
    )
    out = pl.pallas_call(
        _tc_body,
        grid_spec=grid_spec,
        out_shape=jax.ShapeDtypeStruct((_N, _P, _F), jnp.float32),
    )(tl, tt, x3, pt, ft)
    return out.reshape(_B, _T, _P, _F)


# speedup vs baseline: 1.6659x; 1.6659x over previous
import functools

import jax
import jax.numpy as jnp
from jax.experimental import pallas as pl
from jax.experimental.pallas import tpu as pltpu

_B, _T, _P, _F = 16, 50, 200, 128
_N = _B * _T
_G = 32


def _tc_body(tl_sp, tt_sp, x_ref, pt_ref, ft_ref, o_ref):
    g = pl.program_id(0)
    pf = pt_ref[...] + ft_ref[...]
    for r in range(_G):
        idx = tl_sp[g * _G + r] // 3
        te = tt_sp[idx]
        o_ref[r] = x_ref[r] + (pf + te)


@jax.jit
def kernel(x, time_list, time_table, point_table, f_table):
    x3 = x.reshape(_N, _P, _F)
    tl = time_list.reshape(_N).astype(jnp.int32)
    tt = time_table.reshape(-1)
    pt = point_table.reshape(_P, 1)
    ft = f_table.reshape(1, _F)

    grid_spec = pltpu.PrefetchScalarGridSpec(
        num_scalar_prefetch=2,
        grid=(_N // _G,),
        in_specs=[
            pl.BlockSpec((_G, _P, _F), lambda g, tl_sp, tt_sp: (g, 0, 0)),
            pl.BlockSpec((_P, 1), lambda g, tl_sp, tt_sp: (0, 0)),
            pl.BlockSpec((1, _F), lambda g, tl_sp, tt_sp: (0, 0)),
        ],
        out_specs=pl.BlockSpec((_G, _P, _F), lambda g, tl_sp, tt_sp: (g, 0, 0)),
    )
    out = pl.pallas_call(
        _tc_body,
        grid_spec=grid_spec,
        out_shape=jax.ShapeDtypeStruct((_N, _P, _F), jnp.float32),
    )(tl, tt, x3, pt, ft)
    return out.reshape(_B, _T, _P, _F)


# speedup vs baseline: 1.7176x; 1.0310x over previous
import functools

import jax
import jax.numpy as jnp
from jax.experimental import pallas as pl
from jax.experimental.pallas import tpu as pltpu

_B, _T, _P, _F = 16, 50, 200, 128
_N = _B * _T
_G = 50


def _tc_body(tl_sp, tt_sp, x_ref, pt_ref, ft_ref, o_ref):
    g = pl.program_id(0)
    pf = pt_ref[...] + ft_ref[...]
    for r in range(_G):
        idx = tl_sp[g * _G + r] // 3
        te = tt_sp[idx]
        o_ref[r] = x_ref[r] + (pf + te)


@jax.jit
def kernel(x, time_list, time_table, point_table, f_table):
    x3 = x.reshape(_N, _P, _F)
    tl = time_list.reshape(_N).astype(jnp.int32)
    tt = time_table.reshape(-1)
    pt = point_table.reshape(_P, 1)
    ft = f_table.reshape(1, _F)

    grid_spec = pltpu.PrefetchScalarGridSpec(
        num_scalar_prefetch=2,
        grid=(_N // _G,),
        in_specs=[
            pl.BlockSpec((_G, _P, _F), lambda g, tl_sp, tt_sp: (g, 0, 0)),
            pl.BlockSpec((_P, 1), lambda g, tl_sp, tt_sp: (0, 0)),
            pl.BlockSpec((1, _F), lambda g, tl_sp, tt_sp: (0, 0)),
        ],
        out_specs=pl.BlockSpec((_G, _P, _F), lambda g, tl_sp, tt_sp: (g, 0, 0)),
    )
    out = pl.pallas_call(
        _tc_body,
        grid_spec=grid_spec,
        out_shape=jax.ShapeDtypeStruct((_N, _P, _F), jnp.float32),
    )(tl, tt, x3, pt, ft)
    return out.reshape(_B, _T, _P, _F)


# speedup vs baseline: 1.7662x; 1.0283x over previous
import functools

import jax
import jax.numpy as jnp
from jax.experimental import pallas as pl
from jax.experimental.pallas import tpu as pltpu

_B, _T, _P, _F = 16, 50, 200, 128
_N = _B * _T
_G = 100


def _tc_body(tl_sp, tt_sp, x_ref, pt_ref, ft_ref, o_ref):
    g = pl.program_id(0)
    pf = pt_ref[...] + ft_ref[...]
    for r in range(_G):
        idx = tl_sp[g * _G + r] // 3
        te = tt_sp[idx]
        o_ref[r] = x_ref[r] + (pf + te)


@jax.jit
def kernel(x, time_list, time_table, point_table, f_table):
    x3 = x.reshape(_N, _P, _F)
    tl = time_list.reshape(_N).astype(jnp.int32)
    tt = time_table.reshape(-1)
    pt = point_table.reshape(_P, 1)
    ft = f_table.reshape(1, _F)

    grid_spec = pltpu.PrefetchScalarGridSpec(
        num_scalar_prefetch=2,
        grid=(_N // _G,),
        in_specs=[
            pl.BlockSpec((_G, _P, _F), lambda g, tl_sp, tt_sp: (g, 0, 0)),
            pl.BlockSpec((_P, 1), lambda g, tl_sp, tt_sp: (0, 0)),
            pl.BlockSpec((1, _F), lambda g, tl_sp, tt_sp: (0, 0)),
        ],
        out_specs=pl.BlockSpec((_G, _P, _F), lambda g, tl_sp, tt_sp: (g, 0, 0)),
    )
    out = pl.pallas_call(
        _tc_body,
        grid_spec=grid_spec,
        out_shape=jax.ShapeDtypeStruct((_N, _P, _F), jnp.float32),
    )(tl, tt, x3, pt, ft)
    return out.reshape(_B, _T, _P, _F)
